# Initial kernel scaffold; baseline (speedup 1.0000x reference)
import functools

import jax
import jax.numpy as jnp
from jax import lax
from jax.experimental import pallas as pl
from jax.experimental.pallas import tpu as pltpu
from jax.experimental.pallas import tpu_sc as plsc


def _matmul_split(x, w, nsplit, relu):
    M, K = x.shape
    F = w.shape[1] // nsplit
    Mb = 1000

    def body(x_ref, w_ref, o_ref):
        xv = x_ref[...]
        if relu:
            xv = jnp.maximum(xv, 0.0)
        o_ref[0] = jnp.dot(xv, w_ref[...], preferred_element_type=jnp.float32)

    return pl.pallas_call(
        body,
        grid=(nsplit, M // Mb),
        in_specs=[
            pl.BlockSpec((Mb, K), lambda c, i: (i, 0)),
            pl.BlockSpec((K, F), lambda c, i: (0, c)),
        ],
        out_specs=pl.BlockSpec((1, Mb, F), lambda c, i: (c, i, 0)),
        out_shape=jax.ShapeDtypeStruct((nsplit, M, F), jnp.float32),
    )(x, w)


def _spmm_sc(tbl2, row3, col3, adj3, n_nodes):
    FW = tbl2.shape[2]
    T, NB, EB = col3.shape
    RPT = n_nodes // T
    ZB = 125
    NV = FW // 16
    mesh = plsc.VectorSubcoreMesh(core_axis_name="c", subcore_axis_name="s")

    @functools.partial(
        pl.kernel,
        out_type=jax.ShapeDtypeStruct((n_nodes, 2 * FW), jnp.float32),
        mesh=mesh,
        scratch_types=[
            pltpu.VMEM((NB, EB), jnp.int32),
            pltpu.VMEM((NB, EB), jnp.int32),
            pltpu.VMEM((NB, EB), jnp.float32),
            pltpu.VMEM((EB, FW), jnp.float32),
            pltpu.VMEM((ZB, FW), jnp.float32),
            pltpu.VMEM_SHARED((n_nodes, FW), jnp.float32),
        ],
    )
    def k(tbl, row_h, col_h, adj_h, out, colv, rowv, adjv, gbuf, zbuf, acc):
        c = lax.axis_index("c")
        s = lax.axis_index("s")

        pltpu.sync_copy(col_h.at[s], colv)
        pltpu.sync_copy(row_h.at[s], rowv)
        pltpu.sync_copy(adj_h.at[s], adjv)

        zero16 = jnp.zeros((16,), jnp.float32)

        @pl.loop(0, ZB)
        def _(i):
            for v in range(NV):
                zbuf[i, pl.ds(v * 16, 16)] = zero16

        for z in range(RPT // ZB):
            pltpu.sync_copy(zbuf, acc.at[pl.ds(s * RPT + z * ZB, ZB)])
        plsc.subcore_barrier()

        @pl.loop(0, NB)
        def _(b):
            pltpu.sync_copy(tbl.at[c, colv.at[b]], gbuf)
            for g in range(EB // 16):
                a16 = adjv[b, pl.ds(g * 16, 16)]
                for j in range(16):
                    e = g * 16 + j
                    spl = jnp.take(a16, jnp.full((16,), j, jnp.int32),
                                   mode="promise_in_bounds")
                    for v in range(NV):
                        gbuf[e, pl.ds(v * 16, 16)] = (
                            gbuf[e, pl.ds(v * 16, 16)] * spl)
            pltpu.sync_copy(gbuf, acc.at[rowv.at[b]], add=True)

        plsc.subcore_barrier()
        pltpu.sync_copy(
            acc.at[pl.ds(s * RPT, RPT)],
            out.at[pl.ds(s * RPT, RPT), pl.ds(c * FW, FW)],
        )

    return k(tbl2, row3, col3, adj3)


def kernel(x, edge_index, adj_values, W1, W2):
    N = x.shape[0]
    E = adj_values.shape[0]
    T, EB = 16, 80
    NB = E // (T * EB)
    row3 = edge_index[0].reshape(T, NB, EB)
    col3 = edge_index[1].reshape(T, NB, EB)
    adj3 = adj_values.reshape(T, NB, EB)

    P2 = _matmul_split(x, W1, 2, relu=False)
    A = _spmm_sc(P2, row3, col3, adj3, N)
    Q2 = _matmul_split(A, W2, 2, relu=True)
    out = _spmm_sc(Q2, row3, col3, adj3, N)
    return out


# speedup vs baseline: 5.2511x; 5.2511x over previous
import functools

import jax
import jax.numpy as jnp
from jax import lax
from jax.experimental import pallas as pl
from jax.experimental.pallas import tpu as pltpu
from jax.experimental.pallas import tpu_sc as plsc

_T = 16
_EB = 80


def _matmul_split(x, w, nsplit, n_pad):
    M, K = x.shape
    F = w.shape[1] // nsplit
    Mb = 1000
    w_split = jnp.moveaxis(w.reshape(K, nsplit, F), 1, 0)

    def body(x_ref, w_ref, o_ref):
        o_ref[0] = jnp.dot(x_ref[...], w_ref[0],
                           preferred_element_type=jnp.float32)

    return pl.pallas_call(
        body,
        grid=(nsplit, M // Mb),
        in_specs=[
            pl.BlockSpec((Mb, K), lambda c, i: (i, 0)),
            pl.BlockSpec((1, K, F), lambda c, i: (c, 0, 0)),
        ],
        out_specs=pl.BlockSpec((1, Mb, F), lambda c, i: (c, i, 0)),
        out_shape=jax.ShapeDtypeStruct((nsplit, n_pad, F), jnp.float32),
    )(x, w_split)


def _relu_matmul_merge(a2, w, nsplit, n_rows):
    _, n_pad, K2 = a2.shape
    F = w.shape[1] // nsplit
    Mb = 1000
    w_split = jnp.moveaxis(w.reshape(2 * K2, nsplit, F), 1, 0)

    def body(a_ref, w_ref, o_ref):
        a0 = jnp.maximum(a_ref[0], 0.0)
        a1 = jnp.maximum(a_ref[1], 0.0)
        o_ref[0] = (
            jnp.dot(a0, w_ref[0, :K2], preferred_element_type=jnp.float32)
            + jnp.dot(a1, w_ref[0, K2:], preferred_element_type=jnp.float32)
        )

    return pl.pallas_call(
        body,
        grid=(nsplit, n_rows // Mb),
        in_specs=[
            pl.BlockSpec((2, Mb, K2), lambda c, i: (0, i, 0)),
            pl.BlockSpec((1, 2 * K2, F), lambda c, i: (c, 0, 0)),
        ],
        out_specs=pl.BlockSpec((1, Mb, F), lambda c, i: (c, i, 0)),
        out_shape=jax.ShapeDtypeStruct((nsplit, n_pad, F), jnp.float32),
    )(a2, w_split)


def _interleave(o2, n_rows):
    _, n_pad, F = o2.shape
    Mb = 1000

    def body(a_ref, o_ref):
        o_ref[:, :F] = a_ref[0]
        o_ref[:, F:] = a_ref[1]

    return pl.pallas_call(
        body,
        grid=(n_rows // Mb,),
        in_specs=[pl.BlockSpec((2, Mb, F), lambda i: (0, i, 0))],
        out_specs=pl.BlockSpec((Mb, 2 * F), lambda i: (i, 0)),
        out_shape=jax.ShapeDtypeStruct((n_rows, 2 * F), jnp.float32),
    )(o2)


def _spmm_sc(tbl2, row3, col3, adj3):
    n_pad, FW = tbl2.shape[1], tbl2.shape[2]
    T, NB, EB = col3.shape
    RPT = n_pad // T
    ZB = 128
    NV = FW // 16
    mesh = plsc.VectorSubcoreMesh(core_axis_name="c", subcore_axis_name="s")

    @functools.partial(
        pl.kernel,
        out_type=jax.ShapeDtypeStruct((2, n_pad, FW), jnp.float32),
        mesh=mesh,
        compiler_params=pltpu.CompilerParams(use_tc_tiling_on_sc=False),
        scratch_types=[
            pltpu.VMEM((NB, EB), jnp.int32),
            pltpu.VMEM((NB, EB), jnp.int32),
            pltpu.VMEM((NB, EB), jnp.float32),
            pltpu.VMEM((EB, FW), jnp.float32),
            pltpu.VMEM((ZB, FW), jnp.float32),
            pltpu.VMEM_SHARED((n_pad, FW), jnp.float32),
        ],
    )
    def k(tbl, row_h, col_h, adj_h, out, colv, rowv, adjv, gbuf, zbuf, acc):
        c = lax.axis_index("c")
        s = lax.axis_index("s")

        pltpu.sync_copy(col_h.at[s], colv)
        pltpu.sync_copy(row_h.at[s], rowv)
        pltpu.sync_copy(adj_h.at[s], adjv)

        zero16 = jnp.zeros((16,), jnp.float32)

        @pl.loop(0, ZB)
        def _(i):
            for v in range(NV):
                zbuf[i, pl.ds(v * 16, 16)] = zero16

        for z in range(RPT // ZB):
            pltpu.sync_copy(zbuf, acc.at[pl.ds(s * RPT + z * ZB, ZB)])
        plsc.subcore_barrier()

        @pl.loop(0, NB)
        def _(b):
            pltpu.sync_copy(tbl.at[c].at[colv.at[b]], gbuf)
            for g in range(EB // 16):
                a16 = adjv[b, pl.ds(g * 16, 16)]
                for j in range(16):
                    e = g * 16 + j
                    spl = jnp.take_along_axis(
                        a16, jnp.full((16,), j, jnp.int32), axis=0)
                    for v in range(NV):
                        gbuf[e, pl.ds(v * 16, 16)] = (
                            gbuf[e, pl.ds(v * 16, 16)] * spl)
            pltpu.sync_copy(gbuf, acc.at[rowv.at[b]], add=True)

        plsc.subcore_barrier()
        pltpu.sync_copy(
            acc.at[pl.ds(s * RPT, RPT)],
            out.at[c].at[pl.ds(s * RPT, RPT)],
        )

    return k(tbl2, row3, col3, adj3)


def kernel(x, edge_index, adj_values, W1, W2):
    N = x.shape[0]
    E = adj_values.shape[0]
    NB = E // (_T * _EB)
    n_pad = ((N + 128 * _T - 1) // (128 * _T)) * (128 * _T)
    row3 = edge_index[0].reshape(_T, NB, _EB)
    col3 = edge_index[1].reshape(_T, NB, _EB)
    adj3 = adj_values.reshape(_T, NB, _EB)

    P2 = _matmul_split(x, W1, 2, n_pad)
    A2 = _spmm_sc(P2, row3, col3, adj3)
    Q2 = _relu_matmul_merge(A2, W2, 2, N)
    O2 = _spmm_sc(Q2, row3, col3, adj3)
    return _interleave(O2, N)


# speedup vs baseline: 7.0390x; 1.3405x over previous
import functools

import jax
import jax.numpy as jnp
from jax import lax
from jax.experimental import pallas as pl
from jax.experimental.pallas import tpu as pltpu
from jax.experimental.pallas import tpu_sc as plsc

_T = 16
_EB = 80


def _matmul_split(x, w, nsplit, n_pad):
    M, K = x.shape
    F = w.shape[1] // nsplit
    Mb = 1000
    w_split = jnp.moveaxis(w.reshape(K, nsplit, F), 1, 0)

    def body(x_ref, w_ref, o_ref):
        o_ref[0] = jnp.dot(x_ref[...], w_ref[0],
                           preferred_element_type=jnp.float32)

    return pl.pallas_call(
        body,
        grid=(nsplit, M // Mb),
        in_specs=[
            pl.BlockSpec((Mb, K), lambda c, i: (i, 0)),
            pl.BlockSpec((1, K, F), lambda c, i: (c, 0, 0)),
        ],
        out_specs=pl.BlockSpec((1, Mb, F), lambda c, i: (c, i, 0)),
        out_shape=jax.ShapeDtypeStruct((nsplit, n_pad, F), jnp.float32),
    )(x, w_split)


def _relu_matmul_merge(a2, w, nsplit, n_rows):
    _, n_pad, K2 = a2.shape
    F = w.shape[1] // nsplit
    Mb = 1000
    w_split = jnp.moveaxis(w.reshape(2 * K2, nsplit, F), 1, 0)

    def body(a_ref, w_ref, o_ref):
        a0 = jnp.maximum(a_ref[0], 0.0)
        a1 = jnp.maximum(a_ref[1], 0.0)
        o_ref[0] = (
            jnp.dot(a0, w_ref[0, :K2], preferred_element_type=jnp.float32)
            + jnp.dot(a1, w_ref[0, K2:], preferred_element_type=jnp.float32)
        )

    return pl.pallas_call(
        body,
        grid=(nsplit, n_rows // Mb),
        in_specs=[
            pl.BlockSpec((2, Mb, K2), lambda c, i: (0, i, 0)),
            pl.BlockSpec((1, 2 * K2, F), lambda c, i: (c, 0, 0)),
        ],
        out_specs=pl.BlockSpec((1, Mb, F), lambda c, i: (c, i, 0)),
        out_shape=jax.ShapeDtypeStruct((nsplit, n_pad, F), jnp.float32),
    )(a2, w_split)


def _interleave(o2, n_rows):
    _, n_pad, F = o2.shape
    Mb = 1000

    def body(a_ref, o_ref):
        o_ref[:, :F] = a_ref[0]
        o_ref[:, F:] = a_ref[1]

    return pl.pallas_call(
        body,
        grid=(n_rows // Mb,),
        in_specs=[pl.BlockSpec((2, Mb, F), lambda i: (0, i, 0))],
        out_specs=pl.BlockSpec((Mb, 2 * F), lambda i: (i, 0)),
        out_shape=jax.ShapeDtypeStruct((n_rows, 2 * F), jnp.float32),
    )(o2)


def _spmm_sc(tbl2, row3, col3, adj3):
    n_pad, FW = tbl2.shape[1], tbl2.shape[2]
    T, NB, EB = col3.shape
    RPT = n_pad // T
    ZB = 128
    NV = FW // 16
    mesh = plsc.VectorSubcoreMesh(core_axis_name="c", subcore_axis_name="s")

    @functools.partial(
        pl.kernel,
        out_type=jax.ShapeDtypeStruct((2, n_pad, FW), jnp.float32),
        mesh=mesh,
        compiler_params=pltpu.CompilerParams(use_tc_tiling_on_sc=False),
        scratch_types=[
            pltpu.VMEM((NB, EB), jnp.int32),
            pltpu.VMEM((NB, EB), jnp.int32),
            pltpu.VMEM((NB, EB), jnp.float32),
            pltpu.VMEM((2, EB, FW), jnp.float32),
            pltpu.VMEM((ZB, FW), jnp.float32),
            pltpu.VMEM_SHARED((n_pad, FW), jnp.float32),
            pltpu.SemaphoreType.DMA,
            pltpu.SemaphoreType.DMA,
            pltpu.SemaphoreType.DMA,
            pltpu.SemaphoreType.DMA,
        ],
    )
    def k(tbl, row_h, col_h, adj_h, out,
          colv, rowv, adjv, gbuf, zbuf, acc, gsem0, gsem1, ssem0, ssem1):
        c = lax.axis_index("c")
        s = lax.axis_index("s")
        gsems = (gsem0, gsem1)
        ssems = (ssem0, ssem1)

        def gather(b, u):
            return pltpu.make_async_copy(
                tbl.at[c].at[colv.at[b]], gbuf.at[u], gsems[u])

        def scatter(b, u):
            return pltpu.make_async_copy(
                gbuf.at[u], acc.at[rowv.at[b]], ssems[u])

        pltpu.sync_copy(col_h.at[s], colv)
        pltpu.sync_copy(row_h.at[s], rowv)
        pltpu.sync_copy(adj_h.at[s], adjv)

        gather(0, 0).start()

        zero16 = jnp.zeros((16,), jnp.float32)

        @pl.loop(0, ZB)
        def _(i):
            for v in range(NV):
                zbuf[i, pl.ds(v * 16, 16)] = zero16

        for z in range(RPT // ZB):
            pltpu.sync_copy(zbuf, acc.at[pl.ds(s * RPT + z * ZB, ZB)])
        plsc.subcore_barrier()

        @pl.loop(0, NB // 2)
        def _(i):
            for u in range(2):
                b = i * 2 + u
                gather(b, u).wait()
                if u == 0:
                    @pl.when(i > 0)
                    def _():
                        scatter(b - 1, 1).wait()
                    gather(b + 1, 1).start()
                else:
                    scatter(b - 1, 0).wait()

                    @pl.when(b + 1 < NB)
                    def _():
                        gather(b + 1, 0).start()
                for g in range(EB // 16):
                    a16 = adjv[b, pl.ds(g * 16, 16)]
                    for j in range(16):
                        e = g * 16 + j
                        spl = jnp.take_along_axis(
                            a16, jnp.full((16,), j, jnp.int32), axis=0)
                        for v in range(NV):
                            gbuf[u, e, pl.ds(v * 16, 16)] = (
                                gbuf[u, e, pl.ds(v * 16, 16)] * spl)
                scatter(b, u).start(add=True)

        scatter(NB - 1, 1).wait()
        plsc.subcore_barrier()
        pltpu.sync_copy(
            acc.at[pl.ds(s * RPT, RPT)],
            out.at[c].at[pl.ds(s * RPT, RPT)],
        )

    return k(tbl2, row3, col3, adj3)


def kernel(x, edge_index, adj_values, W1, W2):
    N = x.shape[0]
    E = adj_values.shape[0]
    NB = E // (_T * _EB)
    n_pad = ((N + 128 * _T - 1) // (128 * _T)) * (128 * _T)
    row3 = edge_index[0].reshape(_T, NB, _EB)
    col3 = edge_index[1].reshape(_T, NB, _EB)
    adj3 = adj_values.reshape(_T, NB, _EB)

    P2 = _matmul_split(x, W1, 2, n_pad)
    A2 = _spmm_sc(P2, row3, col3, adj3)
    Q2 = _relu_matmul_merge(A2, W2, 2, N)
    O2 = _spmm_sc(Q2, row3, col3, adj3)
    return _interleave(O2, N)


# speedup vs baseline: 9.6431x; 1.3699x over previous
import functools

import jax
import jax.numpy as jnp
from jax import lax
from jax.experimental import pallas as pl
from jax.experimental.pallas import tpu as pltpu
from jax.experimental.pallas import tpu_sc as plsc

_T = 16
_EB = 80


def _matmul_split(x, w, nsplit, n_pad):
    M, K = x.shape
    F = w.shape[1] // nsplit
    Mb = 1000
    w_split = jnp.moveaxis(w.reshape(K, nsplit, F), 1, 0)

    def body(x_ref, w_ref, o_ref):
        o_ref[0] = jnp.dot(x_ref[...], w_ref[0],
                           preferred_element_type=jnp.float32)

    return pl.pallas_call(
        body,
        grid=(nsplit, M // Mb),
        in_specs=[
            pl.BlockSpec((Mb, K), lambda c, i: (i, 0)),
            pl.BlockSpec((1, K, F), lambda c, i: (c, 0, 0)),
        ],
        out_specs=pl.BlockSpec((1, Mb, F), lambda c, i: (c, i, 0)),
        out_shape=jax.ShapeDtypeStruct((nsplit, n_pad, F), jnp.float32),
    )(x, w_split)


def _relu_matmul_merge(a2, w, nsplit, n_rows):
    _, n_pad, K2 = a2.shape
    F = w.shape[1] // nsplit
    Mb = 1000
    w_split = jnp.moveaxis(w.reshape(2 * K2, nsplit, F), 1, 0)

    def body(a_ref, w_ref, o_ref):
        a0 = jnp.maximum(a_ref[0], 0.0)
        a1 = jnp.maximum(a_ref[1], 0.0)
        o_ref[0] = (
            jnp.dot(a0, w_ref[0, :K2], preferred_element_type=jnp.float32)
            + jnp.dot(a1, w_ref[0, K2:], preferred_element_type=jnp.float32)
        )

    return pl.pallas_call(
        body,
        grid=(nsplit, n_rows // Mb),
        in_specs=[
            pl.BlockSpec((2, Mb, K2), lambda c, i: (0, i, 0)),
            pl.BlockSpec((1, 2 * K2, F), lambda c, i: (c, 0, 0)),
        ],
        out_specs=pl.BlockSpec((1, Mb, F), lambda c, i: (c, i, 0)),
        out_shape=jax.ShapeDtypeStruct((nsplit, n_pad, F), jnp.float32),
    )(a2, w_split)


def _interleave(o2, n_rows):
    _, n_pad, F = o2.shape
    Mb = 1000

    def body(a_ref, o_ref):
        o_ref[:, :F] = a_ref[0]
        o_ref[:, F:] = a_ref[1]

    return pl.pallas_call(
        body,
        grid=(n_rows // Mb,),
        in_specs=[pl.BlockSpec((2, Mb, F), lambda i: (0, i, 0))],
        out_specs=pl.BlockSpec((Mb, 2 * F), lambda i: (i, 0)),
        out_shape=jax.ShapeDtypeStruct((n_rows, 2 * F), jnp.float32),
    )(o2)


def _spmm_sc(tbl2, row3, col3, adj3):
    n_pad, FW = tbl2.shape[1], tbl2.shape[2]
    T, NB, EB = col3.shape
    RPT = n_pad // T
    ZB = 128
    NV = FW // 16
    mesh = plsc.VectorSubcoreMesh(core_axis_name="c", subcore_axis_name="s")

    @functools.partial(
        pl.kernel,
        out_type=jax.ShapeDtypeStruct((2, n_pad, FW), jnp.float32),
        mesh=mesh,
        compiler_params=pltpu.CompilerParams(use_tc_tiling_on_sc=False),
        scratch_types=[
            pltpu.VMEM((NB, EB), jnp.int32),
            pltpu.VMEM((NB, EB), jnp.int32),
            pltpu.VMEM((NB, EB), jnp.float32),
            pltpu.VMEM((2, EB, FW), jnp.float32),
            pltpu.VMEM((2, EB, FW), jnp.float32),
            pltpu.VMEM((ZB, FW), jnp.float32),
            pltpu.VMEM_SHARED((n_pad, FW), jnp.float32),
            pltpu.SemaphoreType.DMA,
            pltpu.SemaphoreType.DMA,
            pltpu.SemaphoreType.DMA,
            pltpu.SemaphoreType.DMA,
        ],
    )
    def k(tbl, row_h, col_h, adj_h, out,
          colv, rowv, adjv, gbuf, sbuf, zbuf, acc,
          gsem0, gsem1, ssem0, ssem1):
        c = lax.axis_index("c")
        s = lax.axis_index("s")
        gsems = (gsem0, gsem1)
        ssems = (ssem0, ssem1)

        def gather(b, u):
            return pltpu.make_async_copy(
                tbl.at[c].at[colv.at[b]], gbuf.at[u], gsems[u])

        def scatter(b, u):
            return pltpu.make_async_copy(
                sbuf.at[u], acc.at[rowv.at[b]], ssems[u])

        pltpu.sync_copy(col_h.at[s], colv)
        pltpu.sync_copy(row_h.at[s], rowv)
        pltpu.sync_copy(adj_h.at[s], adjv)

        gather(0, 0).start()
        gather(1, 1).start()

        zero16 = jnp.zeros((16,), jnp.float32)

        @pl.loop(0, ZB)
        def _(i):
            for v in range(NV):
                zbuf[i, pl.ds(v * 16, 16)] = zero16

        for z in range(RPT // ZB):
            pltpu.sync_copy(zbuf, acc.at[pl.ds(s * RPT + z * ZB, ZB)])
        plsc.subcore_barrier()

        @pl.loop(0, NB // 2)
        def _(i):
            for u in range(2):
                b = i * 2 + u
                gather(b, u).wait()

                @pl.when(b >= 2)
                def _():
                    scatter(b - 2, u).wait()
                for g in range(EB // 16):
                    a16 = adjv[b, pl.ds(g * 16, 16)]
                    for j in range(16):
                        e = g * 16 + j
                        spl = jnp.take_along_axis(
                            a16, jnp.full((16,), j, jnp.int32), axis=0)
                        for v in range(NV):
                            sbuf[u, e, pl.ds(v * 16, 16)] = (
                                gbuf[u, e, pl.ds(v * 16, 16)] * spl)

                @pl.when(b + 2 < NB)
                def _():
                    gather(b + 2, u).start()
                scatter(b, u).start(add=True)

        scatter(NB - 2, 0).wait()
        scatter(NB - 1, 1).wait()
        plsc.subcore_barrier()
        pltpu.sync_copy(
            acc.at[pl.ds(s * RPT, RPT)],
            out.at[c].at[pl.ds(s * RPT, RPT)],
        )

    return k(tbl2, row3, col3, adj3)


def kernel(x, edge_index, adj_values, W1, W2):
    N = x.shape[0]
    E = adj_values.shape[0]
    NB = E // (_T * _EB)
    n_pad = ((N + 128 * _T - 1) // (128 * _T)) * (128 * _T)
    row3 = edge_index[0].reshape(_T, NB, _EB)
    col3 = edge_index[1].reshape(_T, NB, _EB)
    adj3 = adj_values.reshape(_T, NB, _EB)

    P2 = _matmul_split(x, W1, 2, n_pad)
    A2 = _spmm_sc(P2, row3, col3, adj3)
    Q2 = _relu_matmul_merge(A2, W2, 2, N)
    O2 = _spmm_sc(Q2, row3, col3, adj3)
    return _interleave(O2, N)


# speedup vs baseline: 9.9088x; 1.0276x over previous
import functools

import jax
import jax.numpy as jnp
from jax import lax
from jax.experimental import pallas as pl
from jax.experimental.pallas import tpu as pltpu
from jax.experimental.pallas import tpu_sc as plsc

_T = 16
_EB = 80


def _matmul_split(x, w, nsplit, n_pad):
    M, K = x.shape
    F = w.shape[1] // nsplit
    Mb = 1000
    w_split = jnp.moveaxis(w.reshape(K, nsplit, F), 1, 0)

    def body(x_ref, w_ref, o_ref):
        o_ref[0] = jnp.dot(x_ref[...], w_ref[0],
                           preferred_element_type=jnp.float32)

    return pl.pallas_call(
        body,
        grid=(nsplit, M // Mb),
        in_specs=[
            pl.BlockSpec((Mb, K), lambda c, i: (i, 0)),
            pl.BlockSpec((1, K, F), lambda c, i: (c, 0, 0)),
        ],
        out_specs=pl.BlockSpec((1, Mb, F), lambda c, i: (c, i, 0)),
        out_shape=jax.ShapeDtypeStruct((nsplit, n_pad, F), jnp.float32),
    )(x, w_split)


def _relu_matmul_merge(a2, w, nsplit, n_rows):
    _, n_pad, K2 = a2.shape
    F = w.shape[1] // nsplit
    Mb = 1000
    w_split = jnp.moveaxis(w.reshape(2 * K2, nsplit, F), 1, 0)

    def body(a_ref, w_ref, o_ref):
        a0 = jnp.maximum(a_ref[0], 0.0)
        a1 = jnp.maximum(a_ref[1], 0.0)
        o_ref[0] = (
            jnp.dot(a0, w_ref[0, :K2], preferred_element_type=jnp.float32)
            + jnp.dot(a1, w_ref[0, K2:], preferred_element_type=jnp.float32)
        )

    return pl.pallas_call(
        body,
        grid=(nsplit, n_rows // Mb),
        in_specs=[
            pl.BlockSpec((2, Mb, K2), lambda c, i: (0, i, 0)),
            pl.BlockSpec((1, 2 * K2, F), lambda c, i: (c, 0, 0)),
        ],
        out_specs=pl.BlockSpec((1, Mb, F), lambda c, i: (c, i, 0)),
        out_shape=jax.ShapeDtypeStruct((nsplit, n_pad, F), jnp.float32),
    )(a2, w_split)


def _interleave(o2, n_rows):
    _, n_pad, F = o2.shape
    Mb = 1000

    def body(a_ref, o_ref):
        o_ref[:, :F] = a_ref[0]
        o_ref[:, F:] = a_ref[1]

    return pl.pallas_call(
        body,
        grid=(n_rows // Mb,),
        in_specs=[pl.BlockSpec((2, Mb, F), lambda i: (0, i, 0))],
        out_specs=pl.BlockSpec((Mb, 2 * F), lambda i: (i, 0)),
        out_shape=jax.ShapeDtypeStruct((n_rows, 2 * F), jnp.float32),
    )(o2)


def _spmm_sc(tbl2, edge_index, adj_values, n_pad):
    FW = tbl2.shape[2]
    EPT = adj_values.shape[0] // _T
    EB = _EB
    NBF = (EPT // EB) & ~1
    TAIL = EPT - NBF * EB
    RPT = n_pad // _T
    ZB = 128
    NV = FW // 16
    mesh = plsc.VectorSubcoreMesh(core_axis_name="c", subcore_axis_name="s")

    @functools.partial(
        pl.kernel,
        out_type=jax.ShapeDtypeStruct((2, n_pad, FW), jnp.float32),
        mesh=mesh,
        compiler_params=pltpu.CompilerParams(use_tc_tiling_on_sc=False),
        scratch_types=[
            pltpu.VMEM((EPT,), jnp.int32),
            pltpu.VMEM((EPT,), jnp.int32),
            pltpu.VMEM((EPT,), jnp.float32),
            pltpu.VMEM((2, EB, FW), jnp.float32),
            pltpu.VMEM((2, EB, FW), jnp.float32),
            pltpu.VMEM((ZB, FW), jnp.float32),
            pltpu.VMEM_SHARED((n_pad, FW), jnp.float32),
            pltpu.SemaphoreType.DMA,
            pltpu.SemaphoreType.DMA,
            pltpu.SemaphoreType.DMA,
            pltpu.SemaphoreType.DMA,
        ],
    )
    def k(tbl, edge_h, adj_h, out,
          colv, rowv, adjv, gbuf, sbuf, zbuf, acc,
          gsem0, gsem1, ssem0, ssem1):
        c = lax.axis_index("c")
        s = lax.axis_index("s")
        gsems = (gsem0, gsem1)
        ssems = (ssem0, ssem1)

        def gather(b, u, n=EB):
            return pltpu.make_async_copy(
                tbl.at[c].at[colv.at[pl.ds(b * EB, n)]],
                gbuf.at[u], gsems[u])

        def scatter(b, u, n=EB):
            return pltpu.make_async_copy(
                sbuf.at[u],
                acc.at[rowv.at[pl.ds(b * EB, n)]], ssems[u])

        def scale(b, u, n=EB):
            for g in range(n // 16):
                a16 = adjv[pl.ds(b * EB + g * 16, 16)]
                for j in range(16):
                    e = g * 16 + j
                    spl = jnp.take_along_axis(
                        a16, jnp.full((16,), j, jnp.int32), axis=0)
                    for v in range(NV):
                        sbuf[u, e, pl.ds(v * 16, 16)] = (
                            gbuf[u, e, pl.ds(v * 16, 16)] * spl)

        base = s * EPT
        pltpu.sync_copy(edge_h.at[1].at[pl.ds(base, EPT)], colv)
        pltpu.sync_copy(edge_h.at[0].at[pl.ds(base, EPT)], rowv)
        pltpu.sync_copy(adj_h.at[pl.ds(base, EPT)], adjv)

        gather(0, 0).start()
        gather(1, 1).start()

        zero16 = jnp.zeros((16,), jnp.float32)

        @pl.loop(0, ZB)
        def _(i):
            for v in range(NV):
                zbuf[i, pl.ds(v * 16, 16)] = zero16

        for z in range(RPT // ZB):
            pltpu.sync_copy(zbuf, acc.at[pl.ds(s * RPT + z * ZB, ZB)])
        plsc.subcore_barrier()

        @pl.loop(0, NBF // 2)
        def _(i):
            for u in range(2):
                b = i * 2 + u
                gather(b, u).wait()

                @pl.when(b >= 2)
                def _():
                    scatter(b - 2, u).wait()

                scale(b, u)

                @pl.when(b + 2 < NBF)
                def _():
                    gather(b + 2, u).start()

                scatter(b, u).start(add=True)

        scatter(NBF - 2, 0).wait()
        scatter(NBF - 1, 1).wait()
        plsc.subcore_barrier()
        pltpu.sync_copy(
            acc.at[pl.ds(s * RPT, RPT)],
            out.at[c].at[pl.ds(s * RPT, RPT)],
        )

    return k(tbl2, edge_index, adj_values)


def kernel(x, edge_index, adj_values, W1, W2):
    N = x.shape[0]
    n_pad = ((N + 128 * _T - 1) // (128 * _T)) * (128 * _T)

    P2 = _matmul_split(x, W1, 2, n_pad)
    A2 = _spmm_sc(P2, edge_index, adj_values, n_pad)
    Q2 = _relu_matmul_merge(A2, W2, 2, N)
    O2 = _spmm_sc(Q2, edge_index, adj_values, n_pad)
    return _interleave(O2, N)


# speedup vs baseline: 9.9097x; 1.0001x over previous
import functools

import jax
import jax.numpy as jnp
from jax import lax
from jax.experimental import pallas as pl
from jax.experimental.pallas import tpu as pltpu
from jax.experimental.pallas import tpu_sc as plsc

_T = 16
_EB = 80
_NR = 2
_LD = 2


def _matmul_split(x, w, nsplit, n_pad):
    M, K = x.shape
    F = w.shape[1] // nsplit
    Mb = 1000
    w_split = jnp.moveaxis(w.reshape(K, nsplit, F), 1, 0)

    def body(x_ref, w_ref, o_ref):
        o_ref[0] = jnp.dot(x_ref[...], w_ref[0],
                           preferred_element_type=jnp.float32)

    return pl.pallas_call(
        body,
        grid=(nsplit, M // Mb),
        in_specs=[
            pl.BlockSpec((Mb, K), lambda c, i: (i, 0)),
            pl.BlockSpec((1, K, F), lambda c, i: (c, 0, 0)),
        ],
        out_specs=pl.BlockSpec((1, Mb, F), lambda c, i: (c, i, 0)),
        out_shape=jax.ShapeDtypeStruct((nsplit, n_pad, F), jnp.float32),
    )(x, w_split)


def _relu_matmul_merge(a2, w, nsplit, n_rows):
    _, n_pad, K2 = a2.shape
    F = w.shape[1] // nsplit
    Mb = 1000
    w_split = jnp.moveaxis(w.reshape(2 * K2, nsplit, F), 1, 0)

    def body(a_ref, w_ref, o_ref):
        a0 = jnp.maximum(a_ref[0], 0.0)
        a1 = jnp.maximum(a_ref[1], 0.0)
        o_ref[0] = (
            jnp.dot(a0, w_ref[0, :K2], preferred_element_type=jnp.float32)
            + jnp.dot(a1, w_ref[0, K2:], preferred_element_type=jnp.float32)
        )

    return pl.pallas_call(
        body,
        grid=(nsplit, n_rows // Mb),
        in_specs=[
            pl.BlockSpec((2, Mb, K2), lambda c, i: (0, i, 0)),
            pl.BlockSpec((1, 2 * K2, F), lambda c, i: (c, 0, 0)),
        ],
        out_specs=pl.BlockSpec((1, Mb, F), lambda c, i: (c, i, 0)),
        out_shape=jax.ShapeDtypeStruct((nsplit, n_pad, F), jnp.float32),
    )(a2, w_split)


def _interleave(o2, n_rows):
    _, n_pad, F = o2.shape
    Mb = 1000

    def body(a_ref, o_ref):
        o_ref[:, :F] = a_ref[0]
        o_ref[:, F:] = a_ref[1]

    return pl.pallas_call(
        body,
        grid=(n_rows // Mb,),
        in_specs=[pl.BlockSpec((2, Mb, F), lambda i: (0, i, 0))],
        out_specs=pl.BlockSpec((Mb, 2 * F), lambda i: (i, 0)),
        out_shape=jax.ShapeDtypeStruct((n_rows, 2 * F), jnp.float32),
    )(o2)


def _spmm_sc(tbl2, edge_index, adj_values, n_pad):
    FW = tbl2.shape[2]
    EPT = adj_values.shape[0] // _T
    EB = _EB
    NBF = EPT // EB
    RPT = n_pad // _T
    ZB = 128
    NV = FW // 16
    mesh = plsc.VectorSubcoreMesh(core_axis_name="c", subcore_axis_name="s")

    @functools.partial(
        pl.kernel,
        out_type=jax.ShapeDtypeStruct((2, n_pad, FW), jnp.float32),
        mesh=mesh,
        compiler_params=pltpu.CompilerParams(use_tc_tiling_on_sc=False),
        scratch_types=[
            pltpu.VMEM((EPT,), jnp.int32),
            pltpu.VMEM((EPT,), jnp.int32),
            pltpu.VMEM((EPT,), jnp.float32),
            pltpu.VMEM((_NR, EB, FW), jnp.float32),
            pltpu.VMEM((_NR, EB, FW), jnp.float32),
            pltpu.VMEM((ZB, FW), jnp.float32),
            pltpu.VMEM_SHARED((n_pad, FW), jnp.float32),
            [pltpu.SemaphoreType.DMA] * _NR,
            [pltpu.SemaphoreType.DMA] * _NR,
        ],
    )
    def k(tbl, edge_h, adj_h, out,
          colv, rowv, adjv, gbuf, sbuf, zbuf, acc, gsems, ssems):
        c = lax.axis_index("c")
        s = lax.axis_index("s")

        def gather(b, u):
            return pltpu.make_async_copy(
                tbl.at[c].at[colv.at[pl.ds(b * EB, EB)]],
                gbuf.at[u], gsems[u])

        def scatter(b, u):
            return pltpu.make_async_copy(
                sbuf.at[u], acc.at[rowv.at[pl.ds(b * EB, EB)]], ssems[u])

        def scale(b, u):
            for g in range(EB // 16):
                a16 = adjv[pl.ds(b * EB + g * 16, 16)]
                for j in range(16):
                    e = g * 16 + j
                    spl = jnp.take_along_axis(
                        a16, jnp.full((16,), j, jnp.int32), axis=0)
                    for v in range(NV):
                        sbuf[u, e, pl.ds(v * 16, 16)] = (
                            gbuf[u, e, pl.ds(v * 16, 16)] * spl)

        base = s * EPT
        pltpu.sync_copy(edge_h.at[1].at[pl.ds(base, EPT)], colv)
        pltpu.sync_copy(edge_h.at[0].at[pl.ds(base, EPT)], rowv)
        pltpu.sync_copy(adj_h.at[pl.ds(base, EPT)], adjv)

        for u in range(_LD):
            gather(u, u).start()

        zero16 = jnp.zeros((16,), jnp.float32)

        @pl.loop(0, ZB)
        def _(i):
            for v in range(NV):
                zbuf[i, pl.ds(v * 16, 16)] = zero16

        for z in range(RPT // ZB):
            pltpu.sync_copy(zbuf, acc.at[pl.ds(s * RPT + z * ZB, ZB)])
        plsc.subcore_barrier()

        @pl.loop(0, NBF // _NR)
        def _(i):
            for u in range(_NR):
                b = i * _NR + u
                gather(b, u).wait()

                if u >= _LD:
                    scatter(b - _LD, u - _LD).wait()
                else:
                    @pl.when(i > 0)
                    def _():
                        scatter(b - _LD, (u - _LD) % _NR).wait()

                scale(b, u)

                bn = b + _LD
                if u < _NR - _LD:
                    gather(bn, (u + _LD) % _NR).start()
                else:
                    @pl.when(bn < NBF)
                    def _():
                        gather(bn, (u + _LD) % _NR).start()

                scatter(b, u).start(add=True)

        for j in range(_LD):
            b = NBF - _LD + j
            scatter(b, b % _NR).wait()
        plsc.subcore_barrier()
        pltpu.sync_copy(
            acc.at[pl.ds(s * RPT, RPT)],
            out.at[c].at[pl.ds(s * RPT, RPT)],
        )

    return k(tbl2, edge_index, adj_values)


def kernel(x, edge_index, adj_values, W1, W2):
    N = x.shape[0]
    n_pad = ((N + 128 * _T - 1) // (128 * _T)) * (128 * _T)

    P2 = _matmul_split(x, W1, 2, n_pad)
    A2 = _spmm_sc(P2, edge_index, adj_values, n_pad)
    Q2 = _relu_matmul_merge(A2, W2, 2, N)
    O2 = _spmm_sc(Q2, edge_index, adj_values, n_pad)
    return _interleave(O2, N)


# speedup vs baseline: 10.2848x; 1.0379x over previous
import functools

import jax
import jax.numpy as jnp
from jax import lax
from jax.experimental import pallas as pl
from jax.experimental.pallas import tpu as pltpu
from jax.experimental.pallas import tpu_sc as plsc

_T = 16
_EB = 80
_NR = 2
_LD = 2


def _matmul_split(x, w, nsplit, n_pad):
    M, K = x.shape
    F = w.shape[1] // nsplit
    Mb = 1000
    w_split = jnp.moveaxis(w.reshape(K, nsplit, F), 1, 0)

    def body(x_ref, w_ref, o_ref):
        xv = x_ref[...]
        for p in range(nsplit):
            o_ref[p] = jnp.dot(xv, w_ref[p],
                               preferred_element_type=jnp.float32)

    return pl.pallas_call(
        body,
        grid=(M // Mb,),
        in_specs=[
            pl.BlockSpec((Mb, K), lambda i: (i, 0)),
            pl.BlockSpec((nsplit, K, F), lambda i: (0, 0, 0)),
        ],
        out_specs=pl.BlockSpec((nsplit, Mb, F), lambda i: (0, i, 0)),
        out_shape=jax.ShapeDtypeStruct((nsplit, n_pad, F), jnp.float32),
    )(x, w_split)


def _relu_matmul_merge(a2, w, nsplit, n_rows):
    _, n_pad, K2 = a2.shape
    F = w.shape[1] // nsplit
    Mb = 1000
    w_split = jnp.moveaxis(w.reshape(2 * K2, nsplit, F), 1, 0)

    def body(a_ref, w_ref, o_ref):
        a0 = jnp.maximum(a_ref[0], 0.0)
        a1 = jnp.maximum(a_ref[1], 0.0)
        for p in range(nsplit):
            o_ref[p] = (
                jnp.dot(a0, w_ref[p, :K2], preferred_element_type=jnp.float32)
                + jnp.dot(a1, w_ref[p, K2:], preferred_element_type=jnp.float32)
            )

    return pl.pallas_call(
        body,
        grid=(n_rows // Mb,),
        in_specs=[
            pl.BlockSpec((2, Mb, K2), lambda i: (0, i, 0)),
            pl.BlockSpec((nsplit, 2 * K2, F), lambda i: (0, 0, 0)),
        ],
        out_specs=pl.BlockSpec((nsplit, Mb, F), lambda i: (0, i, 0)),
        out_shape=jax.ShapeDtypeStruct((nsplit, n_pad, F), jnp.float32),
    )(a2, w_split)


def _interleave(o2, n_rows):
    _, n_pad, F = o2.shape
    Mb = 1000

    def body(a_ref, o_ref):
        o_ref[:, :F] = a_ref[0]
        o_ref[:, F:] = a_ref[1]

    return pl.pallas_call(
        body,
        grid=(n_rows // Mb,),
        in_specs=[pl.BlockSpec((2, Mb, F), lambda i: (0, i, 0))],
        out_specs=pl.BlockSpec((Mb, 2 * F), lambda i: (i, 0)),
        out_shape=jax.ShapeDtypeStruct((n_rows, 2 * F), jnp.float32),
    )(o2)


def _spmm_sc(tbl2, edge_index, adj_values, n_pad):
    FW = tbl2.shape[2]
    EPT = adj_values.shape[0] // _T
    EB = _EB
    NBF = EPT // EB
    RPT = n_pad // _T
    ZB = 128
    NV = FW // 16
    mesh = plsc.VectorSubcoreMesh(core_axis_name="c", subcore_axis_name="s")

    @functools.partial(
        pl.kernel,
        out_type=jax.ShapeDtypeStruct((2, n_pad, FW), jnp.float32),
        mesh=mesh,
        compiler_params=pltpu.CompilerParams(use_tc_tiling_on_sc=False),
        scratch_types=[
            pltpu.VMEM((EPT,), jnp.int32),
            pltpu.VMEM((EPT,), jnp.int32),
            pltpu.VMEM((EPT,), jnp.float32),
            pltpu.VMEM((_NR, EB, FW), jnp.float32),
            pltpu.VMEM((_NR, EB, FW), jnp.float32),
            pltpu.VMEM((ZB, FW), jnp.float32),
            pltpu.VMEM_SHARED((n_pad, FW), jnp.float32),
            [pltpu.SemaphoreType.DMA] * _NR,
            [pltpu.SemaphoreType.DMA] * _NR,
        ],
    )
    def k(tbl, edge_h, adj_h, out,
          colv, rowv, adjv, gbuf, sbuf, zbuf, acc, gsems, ssems):
        c = lax.axis_index("c")
        s = lax.axis_index("s")

        def gather(b, u):
            return pltpu.make_async_copy(
                tbl.at[c].at[colv.at[pl.ds(b * EB, EB)]],
                gbuf.at[u], gsems[u])

        def scatter(b, u):
            return pltpu.make_async_copy(
                sbuf.at[u], acc.at[rowv.at[pl.ds(b * EB, EB)]], ssems[u])

        def scale(b, u):
            for g in range(EB // 16):
                a16 = adjv[pl.ds(b * EB + g * 16, 16)]
                for j in range(16):
                    e = g * 16 + j
                    spl = jnp.take_along_axis(
                        a16, jnp.full((16,), j, jnp.int32), axis=0)
                    for v in range(NV):
                        sbuf[u, e, pl.ds(v * 16, 16)] = (
                            gbuf[u, e, pl.ds(v * 16, 16)] * spl)

        base = s * EPT
        pltpu.sync_copy(edge_h.at[1].at[pl.ds(base, EPT)], colv)
        pltpu.sync_copy(edge_h.at[0].at[pl.ds(base, EPT)], rowv)
        pltpu.sync_copy(adj_h.at[pl.ds(base, EPT)], adjv)

        for u in range(_LD):
            gather(u, u).start()

        zero16 = jnp.zeros((16,), jnp.float32)

        @pl.loop(0, ZB)
        def _(i):
            for v in range(NV):
                zbuf[i, pl.ds(v * 16, 16)] = zero16

        for z in range(RPT // ZB):
            pltpu.sync_copy(zbuf, acc.at[pl.ds(s * RPT + z * ZB, ZB)])
        plsc.subcore_barrier()

        @pl.loop(0, NBF // _NR)
        def _(i):
            for u in range(_NR):
                b = i * _NR + u
                gather(b, u).wait()

                if u >= _LD:
                    scatter(b - _LD, u - _LD).wait()
                else:
                    @pl.when(i > 0)
                    def _():
                        scatter(b - _LD, (u - _LD) % _NR).wait()

                scale(b, u)

                bn = b + _LD
                if u < _NR - _LD:
                    gather(bn, (u + _LD) % _NR).start()
                else:
                    @pl.when(bn < NBF)
                    def _():
                        gather(bn, (u + _LD) % _NR).start()

                scatter(b, u).start(add=True)

        for j in range(_LD):
            b = NBF - _LD + j
            scatter(b, b % _NR).wait()
        plsc.subcore_barrier()
        pltpu.sync_copy(
            acc.at[pl.ds(s * RPT, RPT)],
            out.at[c].at[pl.ds(s * RPT, RPT)],
        )

    return k(tbl2, edge_index, adj_values)


def kernel(x, edge_index, adj_values, W1, W2):
    N = x.shape[0]
    n_pad = ((N + 128 * _T - 1) // (128 * _T)) * (128 * _T)

    P2 = _matmul_split(x, W1, 2, n_pad)
    A2 = _spmm_sc(P2, edge_index, adj_values, n_pad)
    Q2 = _relu_matmul_merge(A2, W2, 2, N)
    O2 = _spmm_sc(Q2, edge_index, adj_values, n_pad)
    return _interleave(O2, N)
